# baseline (speedup 1.0000x reference)
import functools

import numpy as np

import jax
import jax.numpy as jnp
from jax import lax
from jax.experimental import pallas as pl
from jax.experimental.pallas import tpu as pltpu
from jax.experimental.pallas import tpu_sc as plsc

ALPHA = 0.2

_NC, _NS, _L = 2, 16, 16
_NW = _NC * _NS

_NODES_PER_BATCH = 8


def _proj_body(x_ref, w_ref, b_ref, ap_ref, f_ref, lpc_ref):
    f = jnp.dot(x_ref[...], w_ref[...], preferred_element_type=jnp.float32)
    f = f + b_ref[...]
    f_ref[...] = f.astype(jnp.bfloat16)
    lpc_ref[...] = jnp.dot(f, ap_ref[...], preferred_element_type=jnp.float32)


def _project(x, W, b, Ap, bn):
    n, c_in = x.shape
    c_out = W.shape[1]
    na = Ap.shape[1]
    grid = (n // bn,)
    return pl.pallas_call(
        _proj_body,
        grid=grid,
        in_specs=[
            pl.BlockSpec((bn, c_in), lambda i: (i, 0)),
            pl.BlockSpec((c_in, c_out), lambda i: (0, 0)),
            pl.BlockSpec((1, c_out), lambda i: (0, 0)),
            pl.BlockSpec((c_in, na), lambda i: (0, 0)),
        ],
        out_specs=[
            pl.BlockSpec((bn, c_out), lambda i: (i, 0)),
            pl.BlockSpec((bn, na), lambda i: (i, 0)),
        ],
        out_shape=[
            jax.ShapeDtypeStruct((n, c_out), jnp.bfloat16),
            jax.ShapeDtypeStruct((n, na), jnp.float32),
        ],
    )(x, W, b.reshape(1, c_out), Ap)


def _make_sc_kernel(np_, m, h, c_head, c_out, lpc_w):
    npw = np_ // _NW
    nb = _NODES_PER_BATCH
    epb = nb * m
    nbatch = npw // nb
    mesh = plsc.VectorSubcoreMesh(core_axis_name="c", subcore_axis_name="s")

    @functools.partial(
        pl.kernel,
        out_type=jax.ShapeDtypeStruct((np_, c_out), jnp.float32),
        mesh=mesh,
        compiler_params=pltpu.CompilerParams(
            use_tc_tiling_on_sc=False, needs_layout_passes=False),
        scratch_types=[
            pltpu.VMEM((npw * m,), jnp.int32),
            pltpu.VMEM((npw, lpc_w), jnp.float32),
            pltpu.VMEM((2, epb, c_out), jnp.bfloat16),
            pltpu.VMEM((2, epb, lpc_w), jnp.float32),
            pltpu.VMEM((m, h), jnp.float32),
            pltpu.VMEM((2, nb, c_out), jnp.float32),
            pltpu.SemaphoreType.DMA,
            pltpu.SemaphoreType.DMA,
            pltpu.SemaphoreType.DMA,
            pltpu.SemaphoreType.DMA,
            pltpu.SemaphoreType.DMA,
            pltpu.SemaphoreType.DMA,
        ],
    )
    def sc_kernel(feats_hbm, lpc_hbm, nbr_hbm, out_hbm,
                  idx_v, lp_v, nbr_buf, lc_buf, e_buf, out_buf,
                  gsem0, gsem1, lsem0, lsem1, osem0, osem1):
        gsem = (gsem0, gsem1)
        lsem = (lsem0, lsem1)
        osem = (osem0, osem1)
        wid = lax.axis_index("s") * _NC + (1 - lax.axis_index("c"))
        nbase = wid * npw

        pltpu.sync_copy(nbr_hbm.at[pl.ds(nbase * m, npw * m)], idx_v)
        pltpu.sync_copy(lpc_hbm.at[pl.ds(nbase, npw)], lp_v)

        def gather_copies(g, kb):
            iv = idx_v.at[pl.ds(g * epb, epb)]
            return (
                pltpu.make_async_copy(feats_hbm.at[iv], nbr_buf.at[kb], gsem[kb]),
                pltpu.make_async_copy(lpc_hbm.at[iv], lc_buf.at[kb], lsem[kb]),
            )

        def out_copy(g, kb):
            return pltpu.make_async_copy(
                out_buf.at[kb],
                out_hbm.at[pl.ds(nbase + g * nb, nb)],
                osem[kb],
            )

        def start_gather(g, kb):
            c1, c2 = gather_copies(g, kb)
            c1.start()
            c2.start()

        def wait_gather(g, kb):
            c1, c2 = gather_copies(g, kb)
            c1.wait()
            c2.wait()

        def compute_batch(g, kb):
            row0 = g * nb

            def node_body(i, _):
                erow = i * m
                lp_vec = lp_v[row0 + i, pl.ds(0, h)]

                def p1(mm, mx):
                    lvec = lp_vec + lc_buf[kb, erow + mm, pl.ds(h, h)]
                    lvec = jnp.maximum(lvec, ALPHA * lvec)
                    e_buf[mm, :] = lvec
                    return jnp.maximum(mx, lvec)

                mx = lax.fori_loop(
                    0, m, p1, jnp.full((h,), -jnp.inf, jnp.float32))

                def p2(mm, s):
                    e = jnp.exp(e_buf[mm, :] - mx)
                    e_buf[mm, :] = e
                    return s + e

                s = lax.fori_loop(0, m, p2, jnp.zeros((h,), jnp.float32))
                inv = 1.0 / s

                def p3(mm, accs):
                    r = erow + mm
                    ev = e_buf[mm, :]
                    new = list(accs)
                    for j in range(h // 2):
                        chunk = nbr_buf[kb, r, pl.ds(2 * j * c_head,
                                                     2 * c_head)]
                        ca, cb = plsc.unpack(
                            chunk, format=plsc.PackFormat.INTERLEAVED,
                            preferred_element_type=jnp.float32)
                        new[2 * j] = new[2 * j] + ev[2 * j] * ca
                        new[2 * j + 1] = new[2 * j + 1] + ev[2 * j + 1] * cb
                    return tuple(new)

                accs = lax.fori_loop(
                    0, m, p3,
                    tuple(jnp.zeros((c_head,), jnp.float32) for _ in range(h)))
                for hh in range(h):
                    out_buf[kb, i, pl.ds(hh * c_head, c_head)] = (
                        accs[hh] * inv[hh])
                return 0

            lax.fori_loop(0, nb, node_body, 0)

        start_gather(0, 0)

        def pair_body(p, _):
            for kb in range(2):
                g = p * 2 + kb

                @pl.when(g + 1 < nbatch)
                def _():
                    start_gather(g + 1, 1 - kb)

                wait_gather(g, kb)

                @pl.when(g >= 2)
                def _():
                    out_copy(g - 2, kb).wait()

                compute_batch(g, kb)
                out_copy(g, kb).start()
            return 0

        lax.fori_loop(0, nbatch // 2, pair_body, 0)

        out_copy(nbatch - 2, 0).wait()
        out_copy(nbatch - 1, 1).wait()

    return sc_kernel


def kernel(node_feats, nbr_idx, W, b, a):
    n, c_in = node_feats.shape
    m = nbr_idx.shape[1]
    c_out = W.shape[1]
    h = a.shape[0]
    c_head = a.shape[1] // 2

    chunk = _NW * _NODES_PER_BATCH
    np_ = ((n + chunk - 1) // chunk) * chunk
    x = jnp.pad(node_feats, ((0, np_ - n), (0, 0)))
    nbr = jnp.pad(nbr_idx.astype(jnp.int32), ((0, np_ - n), (0, 0)))

    eye = jnp.eye(h, dtype=jnp.float32)
    a1 = a[:, :c_head]
    a2 = a[:, c_head:]
    A1p = (a1[:, :, None] * eye[:, None, :]).reshape(h * c_head, h)
    A2p = (a2[:, :, None] * eye[:, None, :]).reshape(h * c_head, h)
    Ap = jnp.concatenate([A1p, A2p], axis=1)

    q = np.arange(c_out)
    jj, rr = q // (2 * c_head), q % (2 * c_head)
    kk, odd = rr // 2, rr % 2
    perm = 2 * c_head * jj + kk + c_head * odd
    W_perm = W[:, perm]
    b_perm = b[perm]
    Ap_perm = Ap[perm, :]

    feats, lpc = _project(x, W_perm, b_perm, Ap_perm,
                          bn=np_ // 10 if np_ % 10 == 0 else np_)

    sc = _make_sc_kernel(np_, m, h, c_head, c_out, 2 * h)
    out = sc(feats, lpc, nbr.reshape(-1))
    return out[:n]


# speedup vs baseline: 1.6442x; 1.6442x over previous
import functools

import numpy as np

import jax
import jax.numpy as jnp
from jax import lax
from jax.experimental import pallas as pl
from jax.experimental.pallas import tpu as pltpu
from jax.experimental.pallas import tpu_sc as plsc

ALPHA = 0.2

_NC, _NS, _L = 2, 16, 16
_NW = _NC * _NS

_NODES_PER_BATCH = 8


def _proj_body(x_ref, w_ref, b_ref, ap_ref, f_ref, lpc_ref):
    f = jnp.dot(x_ref[...], w_ref[...], preferred_element_type=jnp.float32)
    f = f + b_ref[...]
    f_ref[...] = f.astype(jnp.bfloat16)
    lpc_ref[...] = jnp.dot(f, ap_ref[...], preferred_element_type=jnp.float32)


def _project(x, W, b, Ap, bn):
    n, c_in = x.shape
    c_out = W.shape[1]
    na = Ap.shape[1]
    grid = (n // bn,)
    return pl.pallas_call(
        _proj_body,
        grid=grid,
        in_specs=[
            pl.BlockSpec((bn, c_in), lambda i: (i, 0)),
            pl.BlockSpec((c_in, c_out), lambda i: (0, 0)),
            pl.BlockSpec((1, c_out), lambda i: (0, 0)),
            pl.BlockSpec((c_in, na), lambda i: (0, 0)),
        ],
        out_specs=[
            pl.BlockSpec((bn, c_out), lambda i: (i, 0)),
            pl.BlockSpec((bn, na), lambda i: (i, 0)),
        ],
        out_shape=[
            jax.ShapeDtypeStruct((n, c_out), jnp.bfloat16),
            jax.ShapeDtypeStruct((n, na), jnp.float32),
        ],
    )(x, W, b.reshape(1, c_out), Ap)


def _make_sc_kernel(np_, m, h, c_head, c_out, lpc_w):
    npw = np_ // _NW
    nb = _NODES_PER_BATCH
    epb = nb * m
    nbatch = npw // nb
    mesh = plsc.VectorSubcoreMesh(core_axis_name="c", subcore_axis_name="s")

    @functools.partial(
        pl.kernel,
        out_type=jax.ShapeDtypeStruct((np_, c_out), jnp.float32),
        mesh=mesh,
        compiler_params=pltpu.CompilerParams(
            use_tc_tiling_on_sc=False, needs_layout_passes=False),
        scratch_types=[
            pltpu.VMEM((npw * m,), jnp.int32),
            pltpu.VMEM((npw, lpc_w), jnp.float32),
            pltpu.VMEM((2, epb, c_out), jnp.bfloat16),
            pltpu.VMEM((2, epb, lpc_w), jnp.float32),
            pltpu.VMEM((m, h), jnp.float32),
            pltpu.VMEM((2, nb, c_out), jnp.float32),
            pltpu.SemaphoreType.DMA,
            pltpu.SemaphoreType.DMA,
            pltpu.SemaphoreType.DMA,
            pltpu.SemaphoreType.DMA,
            pltpu.SemaphoreType.DMA,
            pltpu.SemaphoreType.DMA,
        ],
    )
    def sc_kernel(feats_hbm, lpc_hbm, nbr_hbm, out_hbm,
                  idx_v, lp_v, nbr_buf, lc_buf, e_buf, out_buf,
                  gsem0, gsem1, lsem0, lsem1, osem0, osem1):
        gsem = (gsem0, gsem1)
        lsem = (lsem0, lsem1)
        osem = (osem0, osem1)
        wid = lax.axis_index("s") * _NC + lax.axis_index("c")
        nbase = wid * npw

        pltpu.sync_copy(nbr_hbm.at[pl.ds(nbase * m, npw * m)], idx_v)
        pltpu.sync_copy(lpc_hbm.at[pl.ds(nbase, npw)], lp_v)

        def gather_copies(g, kb):
            iv = idx_v.at[pl.ds(g * epb, epb)]
            return (
                pltpu.make_async_copy(feats_hbm.at[iv], nbr_buf.at[kb], gsem[kb]),
                pltpu.make_async_copy(lpc_hbm.at[iv], lc_buf.at[kb], lsem[kb]),
            )

        def out_copy(g, kb):
            return pltpu.make_async_copy(
                out_buf.at[kb],
                out_hbm.at[pl.ds(nbase + g * nb, nb)],
                osem[kb],
            )

        def start_gather(g, kb):
            c1, c2 = gather_copies(g, kb)
            c1.start()
            c2.start()

        def wait_gather(g, kb):
            c1, c2 = gather_copies(g, kb)
            c1.wait()
            c2.wait()

        def compute_batch(g, kb):
            row0 = g * nb

            def node_body(i, _):
                erow = i * m
                lp_vec = lp_v[row0 + i, pl.ds(0, h)]

                def p1(mm, mx):
                    lvec = lp_vec + lc_buf[kb, erow + mm, pl.ds(h, h)]
                    lvec = jnp.maximum(lvec, ALPHA * lvec)
                    e_buf[mm, :] = lvec
                    return jnp.maximum(mx, lvec)

                mx = lax.fori_loop(
                    0, m, p1, jnp.full((h,), -jnp.inf, jnp.float32))

                def p2(mm, s):
                    e = jnp.exp(e_buf[mm, :] - mx)
                    e_buf[mm, :] = e
                    return s + e

                s = lax.fori_loop(0, m, p2, jnp.zeros((h,), jnp.float32))
                inv = 1.0 / s

                def p3(mm, accs):
                    r = erow + mm
                    ev = e_buf[mm, :]
                    new = list(accs)
                    for j in range(h // 2):
                        chunk = nbr_buf[kb, r, pl.ds(2 * j * c_head,
                                                     2 * c_head)]
                        ca, cb = plsc.unpack(
                            chunk, format=plsc.PackFormat.INTERLEAVED,
                            preferred_element_type=jnp.float32)
                        new[2 * j] = new[2 * j] + ev[2 * j] * ca
                        new[2 * j + 1] = new[2 * j + 1] + ev[2 * j + 1] * cb
                    return tuple(new)

                accs = lax.fori_loop(
                    0, m, p3,
                    tuple(jnp.zeros((c_head,), jnp.float32) for _ in range(h)))
                for hh in range(h):
                    out_buf[kb, i, pl.ds(hh * c_head, c_head)] = (
                        accs[hh] * inv[hh])
                return 0

            lax.fori_loop(0, nb, node_body, 0)

        start_gather(0, 0)

        def pair_body(p, _):
            for kb in range(2):
                g = p * 2 + kb

                @pl.when(g + 1 < nbatch)
                def _():
                    start_gather(g + 1, 1 - kb)

                wait_gather(g, kb)

                @pl.when(g >= 2)
                def _():
                    out_copy(g - 2, kb).wait()

                compute_batch(g, kb)
                out_copy(g, kb).start()
            return 0

        lax.fori_loop(0, nbatch // 2, pair_body, 0)

        out_copy(nbatch - 2, 0).wait()
        out_copy(nbatch - 1, 1).wait()

    return sc_kernel


def kernel(node_feats, nbr_idx, W, b, a):
    n, c_in = node_feats.shape
    m = nbr_idx.shape[1]
    c_out = W.shape[1]
    h = a.shape[0]
    c_head = a.shape[1] // 2

    chunk = _NW * _NODES_PER_BATCH
    np_ = ((n + chunk - 1) // chunk) * chunk
    x = jnp.pad(node_feats, ((0, np_ - n), (0, 0)))
    npad = np_ - n
    pad_idx = (jnp.arange(npad * m, dtype=jnp.int32) % n).reshape(npad, m)
    nbr = jnp.concatenate([nbr_idx.astype(jnp.int32), pad_idx], axis=0)

    eye = jnp.eye(h, dtype=jnp.float32)
    a1 = a[:, :c_head]
    a2 = a[:, c_head:]
    A1p = (a1[:, :, None] * eye[:, None, :]).reshape(h * c_head, h)
    A2p = (a2[:, :, None] * eye[:, None, :]).reshape(h * c_head, h)
    Ap = jnp.concatenate([A1p, A2p], axis=1)

    q = np.arange(c_out)
    jj, rr = q // (2 * c_head), q % (2 * c_head)
    kk, odd = rr // 2, rr % 2
    perm = 2 * c_head * jj + kk + c_head * odd
    W_perm = W[:, perm]
    b_perm = b[perm]
    Ap_perm = Ap[perm, :]

    feats, lpc = _project(x, W_perm, b_perm, Ap_perm,
                          bn=np_ // 10 if np_ % 10 == 0 else np_)

    sc = _make_sc_kernel(np_, m, h, c_head, c_out, 2 * h)
    out = sc(feats, lpc, nbr.reshape(-1))
    return out[:n]


# speedup vs baseline: 2.0020x; 1.2176x over previous
import functools

import numpy as np

import jax
import jax.numpy as jnp
from jax import lax
from jax.experimental import pallas as pl
from jax.experimental.pallas import tpu as pltpu
from jax.experimental.pallas import tpu_sc as plsc

ALPHA = 0.2

_NC, _NS, _L = 2, 16, 16
_NW = _NC * _NS

_NODES_PER_BATCH = 8


def _proj_body(x_ref, w_ref, b_ref, ap_ref, f_ref, lpc_ref):
    f = jnp.dot(x_ref[...], w_ref[...], preferred_element_type=jnp.float32)
    f = f + b_ref[...]
    f_ref[...] = f.astype(jnp.bfloat16)
    lpc_ref[...] = jnp.dot(f, ap_ref[...], preferred_element_type=jnp.float32)


def _project(x, W, b, Ap, bn):
    n, c_in = x.shape
    c_out = W.shape[1]
    na = Ap.shape[1]
    grid = (n // bn,)
    return pl.pallas_call(
        _proj_body,
        grid=grid,
        in_specs=[
            pl.BlockSpec((bn, c_in), lambda i: (i, 0)),
            pl.BlockSpec((c_in, c_out), lambda i: (0, 0)),
            pl.BlockSpec((1, c_out), lambda i: (0, 0)),
            pl.BlockSpec((c_in, na), lambda i: (0, 0)),
        ],
        out_specs=[
            pl.BlockSpec((bn, c_out), lambda i: (i, 0)),
            pl.BlockSpec((bn, na), lambda i: (i, 0)),
        ],
        out_shape=[
            jax.ShapeDtypeStruct((n, c_out), jnp.bfloat16),
            jax.ShapeDtypeStruct((n, na), jnp.float32),
        ],
    )(x, W, b.reshape(1, c_out), Ap)


def _make_sc_kernel(np_, m, h, c_head, c_out, lpc_w):
    npw = np_ // _NW
    nb = _NODES_PER_BATCH
    epb = nb * m
    nbatch = npw // nb
    mesh = plsc.VectorSubcoreMesh(core_axis_name="c", subcore_axis_name="s")

    @functools.partial(
        pl.kernel,
        out_type=jax.ShapeDtypeStruct((np_, c_out), jnp.float32),
        mesh=mesh,
        compiler_params=pltpu.CompilerParams(
            use_tc_tiling_on_sc=False, needs_layout_passes=False),
        scratch_types=[
            pltpu.VMEM((npw * m,), jnp.int32),
            pltpu.VMEM((npw, lpc_w), jnp.float32),
            pltpu.VMEM((2, epb, c_out), jnp.bfloat16),
            pltpu.VMEM((2, epb, lpc_w), jnp.float32),
            pltpu.VMEM((2, nb, c_out), jnp.float32),
            pltpu.SemaphoreType.DMA,
            pltpu.SemaphoreType.DMA,
            pltpu.SemaphoreType.DMA,
            pltpu.SemaphoreType.DMA,
            pltpu.SemaphoreType.DMA,
            pltpu.SemaphoreType.DMA,
        ],
    )
    def sc_kernel(feats_hbm, lpc_hbm, nbr_hbm, out_hbm,
                  idx_v, lp_v, nbr_buf, lc_buf, out_buf,
                  gsem0, gsem1, lsem0, lsem1, osem0, osem1):
        gsem = (gsem0, gsem1)
        lsem = (lsem0, lsem1)
        osem = (osem0, osem1)
        wid = lax.axis_index("s") * _NC + lax.axis_index("c")
        nbase = wid * npw

        pltpu.sync_copy(nbr_hbm.at[pl.ds(nbase * m, npw * m)], idx_v)
        pltpu.sync_copy(lpc_hbm.at[pl.ds(nbase, npw)], lp_v)

        def gather_copies(g, kb):
            iv = idx_v.at[pl.ds(g * epb, epb)]
            return (
                pltpu.make_async_copy(feats_hbm.at[iv], nbr_buf.at[kb], gsem[kb]),
                pltpu.make_async_copy(lpc_hbm.at[iv], lc_buf.at[kb], lsem[kb]),
            )

        def out_copy(g, kb):
            return pltpu.make_async_copy(
                out_buf.at[kb],
                out_hbm.at[pl.ds(nbase + g * nb, nb)],
                osem[kb],
            )

        def start_gather(g, kb):
            c1, c2 = gather_copies(g, kb)
            c1.start()
            c2.start()

        def wait_gather(g, kb):
            c1, c2 = gather_copies(g, kb)
            c1.wait()
            c2.wait()

        def compute_batch(g, kb):
            row0 = g * nb

            def node_body(i, _):
                erow = i * m
                lp_vec = lp_v[row0 + i, pl.ds(0, h)]

                lvecs = []
                for mm in range(m):
                    lvec = lp_vec + lc_buf[kb, erow + mm, pl.ds(h, h)]
                    lvecs.append(jnp.maximum(lvec, ALPHA * lvec))
                mx = lvecs[0]
                for mm in range(1, m):
                    mx = jnp.maximum(mx, lvecs[mm])
                evs = [jnp.exp(lv - mx) for lv in lvecs]
                s = evs[0]
                for mm in range(1, m):
                    s = s + evs[mm]
                inv = 1.0 / s

                accs = [jnp.zeros((c_head,), jnp.float32) for _ in range(h)]
                for mm in range(m):
                    r = erow + mm
                    ev = evs[mm]
                    for j in range(h // 2):
                        chunk = nbr_buf[kb, r, pl.ds(2 * j * c_head,
                                                     2 * c_head)]
                        ca, cb = plsc.unpack(
                            chunk, format=plsc.PackFormat.INTERLEAVED,
                            preferred_element_type=jnp.float32)
                        accs[2 * j] = accs[2 * j] + ev[2 * j] * ca
                        accs[2 * j + 1] = accs[2 * j + 1] + ev[2 * j + 1] * cb
                for hh in range(h):
                    out_buf[kb, i, pl.ds(hh * c_head, c_head)] = (
                        accs[hh] * inv[hh])
                return 0

            lax.fori_loop(0, nb, node_body, 0)

        start_gather(0, 0)

        def pair_body(p, _):
            for kb in range(2):
                g = p * 2 + kb

                @pl.when(g + 1 < nbatch)
                def _():
                    start_gather(g + 1, 1 - kb)

                wait_gather(g, kb)

                @pl.when(g >= 2)
                def _():
                    out_copy(g - 2, kb).wait()

                compute_batch(g, kb)
                out_copy(g, kb).start()
            return 0

        lax.fori_loop(0, nbatch // 2, pair_body, 0)

        out_copy(nbatch - 2, 0).wait()
        out_copy(nbatch - 1, 1).wait()

    return sc_kernel


def kernel(node_feats, nbr_idx, W, b, a):
    n, c_in = node_feats.shape
    m = nbr_idx.shape[1]
    c_out = W.shape[1]
    h = a.shape[0]
    c_head = a.shape[1] // 2

    chunk = _NW * _NODES_PER_BATCH
    np_ = ((n + chunk - 1) // chunk) * chunk
    x = jnp.pad(node_feats, ((0, np_ - n), (0, 0)))
    npad = np_ - n
    pad_idx = (jnp.arange(npad * m, dtype=jnp.int32) % n).reshape(npad, m)
    nbr = jnp.concatenate([nbr_idx.astype(jnp.int32), pad_idx], axis=0)

    eye = jnp.eye(h, dtype=jnp.float32)
    a1 = a[:, :c_head]
    a2 = a[:, c_head:]
    A1p = (a1[:, :, None] * eye[:, None, :]).reshape(h * c_head, h)
    A2p = (a2[:, :, None] * eye[:, None, :]).reshape(h * c_head, h)
    Ap = jnp.concatenate([A1p, A2p], axis=1)

    q = np.arange(c_out)
    jj, rr = q // (2 * c_head), q % (2 * c_head)
    kk, odd = rr // 2, rr % 2
    perm = 2 * c_head * jj + kk + c_head * odd
    W_perm = W[:, perm]
    b_perm = b[perm]
    Ap_perm = Ap[perm, :]

    feats, lpc = _project(x, W_perm, b_perm, Ap_perm,
                          bn=np_ // 10 if np_ % 10 == 0 else np_)

    sc = _make_sc_kernel(np_, m, h, c_head, c_out, 2 * h)
    out = sc(feats, lpc, nbr.reshape(-1))
    return out[:n]


# speedup vs baseline: 2.2494x; 1.1236x over previous
import functools

import numpy as np

import jax
import jax.numpy as jnp
from jax import lax
from jax.experimental import pallas as pl
from jax.experimental.pallas import tpu as pltpu
from jax.experimental.pallas import tpu_sc as plsc

ALPHA = 0.2

_NC, _NS, _L = 2, 16, 16
_NW = _NC * _NS

_NODES_PER_BATCH = 8


def _proj_body(x_ref, w_ref, b_ref, ap_ref, f_ref, lpc_ref):
    f = jnp.dot(x_ref[...], w_ref[...], preferred_element_type=jnp.float32)
    f = f + b_ref[...]
    f_ref[...] = f.astype(jnp.bfloat16)
    lpc_ref[...] = jnp.dot(f, ap_ref[...], preferred_element_type=jnp.float32)


def _project(x, W, b, Ap, bn):
    n, c_in = x.shape
    c_out = W.shape[1]
    na = Ap.shape[1]
    grid = (n // bn,)
    return pl.pallas_call(
        _proj_body,
        grid=grid,
        in_specs=[
            pl.BlockSpec((bn, c_in), lambda i: (i, 0)),
            pl.BlockSpec((c_in, c_out), lambda i: (0, 0)),
            pl.BlockSpec((1, c_out), lambda i: (0, 0)),
            pl.BlockSpec((c_in, na), lambda i: (0, 0)),
        ],
        out_specs=[
            pl.BlockSpec((bn, c_out), lambda i: (i, 0)),
            pl.BlockSpec((bn, na), lambda i: (i, 0)),
        ],
        out_shape=[
            jax.ShapeDtypeStruct((n, c_out), jnp.bfloat16),
            jax.ShapeDtypeStruct((n, na), jnp.float32),
        ],
    )(x, W, b.reshape(1, c_out), Ap)


def _make_sc_kernel(n, npw, m, h, c_head, c_out, lpc_w):
    nb = _NODES_PER_BATCH
    epb = nb * m
    nbatch = npw // nb
    mesh = plsc.VectorSubcoreMesh(core_axis_name="c", subcore_axis_name="s")

    @functools.partial(
        pl.kernel,
        out_type=jax.ShapeDtypeStruct((n, c_out), jnp.float32),
        mesh=mesh,
        compiler_params=pltpu.CompilerParams(
            use_tc_tiling_on_sc=False, needs_layout_passes=False),
        scratch_types=[
            pltpu.VMEM((npw * m,), jnp.int32),
            pltpu.VMEM((npw, lpc_w), jnp.float32),
            pltpu.VMEM((2, epb, c_out), jnp.bfloat16),
            pltpu.VMEM((2, epb, lpc_w), jnp.float32),
            pltpu.VMEM((2, nb, c_out), jnp.float32),
            pltpu.SemaphoreType.DMA,
            pltpu.SemaphoreType.DMA,
            pltpu.SemaphoreType.DMA,
            pltpu.SemaphoreType.DMA,
            pltpu.SemaphoreType.DMA,
            pltpu.SemaphoreType.DMA,
        ],
    )
    def sc_kernel(feats_hbm, lpc_hbm, nbr_hbm, out_hbm,
                  idx_v, lp_v, nbr_buf, lc_buf, out_buf,
                  gsem0, gsem1, lsem0, lsem1, osem0, osem1):
        gsem = (gsem0, gsem1)
        lsem = (lsem0, lsem1)
        osem = (osem0, osem1)
        wid = lax.axis_index("s") * _NC + lax.axis_index("c")
        nbase = jnp.minimum(wid * npw, n - npw)

        pltpu.sync_copy(nbr_hbm.at[pl.ds(nbase * m, npw * m)], idx_v)
        pltpu.sync_copy(lpc_hbm.at[pl.ds(nbase, npw)], lp_v)

        def gather_copies(g, kb):
            iv = idx_v.at[pl.ds(g * epb, epb)]
            return (
                pltpu.make_async_copy(feats_hbm.at[iv], nbr_buf.at[kb], gsem[kb]),
                pltpu.make_async_copy(lpc_hbm.at[iv], lc_buf.at[kb], lsem[kb]),
            )

        def out_copy(g, kb):
            return pltpu.make_async_copy(
                out_buf.at[kb],
                out_hbm.at[pl.ds(nbase + g * nb, nb)],
                osem[kb],
            )

        def start_gather(g, kb):
            c1, c2 = gather_copies(g, kb)
            c1.start()
            c2.start()

        def wait_gather(g, kb):
            c1, c2 = gather_copies(g, kb)
            c1.wait()
            c2.wait()

        def compute_batch(g, kb):
            row0 = g * nb

            def node_body(i, _):
                erow = i * m
                lp_vec = lp_v[row0 + i, pl.ds(0, h)]

                lvecs = []
                for mm in range(m):
                    lvec = lp_vec + lc_buf[kb, erow + mm, pl.ds(h, h)]
                    lvecs.append(jnp.maximum(lvec, ALPHA * lvec))
                mx = lvecs[0]
                for mm in range(1, m):
                    mx = jnp.maximum(mx, lvecs[mm])
                evs = [jnp.exp(lv - mx) for lv in lvecs]
                s = evs[0]
                for mm in range(1, m):
                    s = s + evs[mm]
                inv = 1.0 / s

                accs = [jnp.zeros((c_head,), jnp.float32) for _ in range(h)]
                for mm in range(m):
                    r = erow + mm
                    ev = evs[mm]
                    for j in range(h // 2):
                        chunk = nbr_buf[kb, r, pl.ds(2 * j * c_head,
                                                     2 * c_head)]
                        ca, cb = plsc.unpack(
                            chunk, format=plsc.PackFormat.INTERLEAVED,
                            preferred_element_type=jnp.float32)
                        accs[2 * j] = accs[2 * j] + ev[2 * j] * ca
                        accs[2 * j + 1] = accs[2 * j + 1] + ev[2 * j + 1] * cb
                for hh in range(h):
                    out_buf[kb, i, pl.ds(hh * c_head, c_head)] = (
                        accs[hh] * inv[hh])
                return 0

            lax.fori_loop(0, nb, node_body, 0)

        start_gather(0, 0)

        def pair_body(p, _):
            for kb in range(2):
                g = p * 2 + kb

                @pl.when(g + 1 < nbatch)
                def _():
                    start_gather(g + 1, 1 - kb)

                wait_gather(g, kb)

                @pl.when(g >= 2)
                def _():
                    out_copy(g - 2, kb).wait()

                compute_batch(g, kb)
                out_copy(g, kb).start()
            return 0

        lax.fori_loop(0, nbatch // 2, pair_body, 0)

        out_copy(nbatch - 2, 0).wait()
        out_copy(nbatch - 1, 1).wait()

    return sc_kernel


def kernel(node_feats, nbr_idx, W, b, a):
    n, c_in = node_feats.shape
    m = nbr_idx.shape[1]
    c_out = W.shape[1]
    h = a.shape[0]
    c_head = a.shape[1] // 2

    npw = -(-n // (_NW * _NODES_PER_BATCH)) * _NODES_PER_BATCH
    nbr = nbr_idx.astype(jnp.int32)

    eye = jnp.eye(h, dtype=jnp.float32)
    a1 = a[:, :c_head]
    a2 = a[:, c_head:]
    A1p = (a1[:, :, None] * eye[:, None, :]).reshape(h * c_head, h)
    A2p = (a2[:, :, None] * eye[:, None, :]).reshape(h * c_head, h)
    Ap = jnp.concatenate([A1p, A2p], axis=1)

    q = np.arange(c_out)
    jj, rr = q // (2 * c_head), q % (2 * c_head)
    kk, odd = rr // 2, rr % 2
    perm = 2 * c_head * jj + kk + c_head * odd
    W_perm = W[:, perm]
    b_perm = b[perm]
    Ap_perm = Ap[perm, :]

    bn = next(d for d in (2048, 2000, 1024, 1008, 512, 400, 256, 128, 80,
                          64, 48, 32, 16) if n % d == 0)
    feats, lpc = _project(node_feats, W_perm, b_perm, Ap_perm, bn=bn)

    sc = _make_sc_kernel(n, npw, m, h, c_head, c_out, 2 * h)
    return sc(feats, lpc, nbr.reshape(-1))
